# baseline (speedup 1.0000x reference)
import functools

import jax
import jax.numpy as jnp
from jax import lax
from jax.experimental import pallas as pl
from jax.experimental.pallas import tpu as pltpu
from jax.experimental.pallas import tpu_sc as plsc

NC, NS = 2, 16
NW = 28
CELLS = 12544
CELLS_W = CELLS // NW
WORDS_W = CELLS_W * 10
PWORDS_W = CELLS_W // 2 * 30
TWORDS_W = CELLS_W * 30
HALF = CELLS // 2 * 20
GROUPS = CELLS_W // 16
STEP = 1.0 / 7

_mesh = plsc.VectorSubcoreMesh(
    core_axis_name="c", subcore_axis_name="s", num_cores=NC, num_subcores=NS
)


def _full(v):
    return jnp.full((16,), v, jnp.int32)


def _copy_half(src_vmem, dst_vmem):
    iota = lax.iota(jnp.int32, 16)
    srcs = tuple(
        (lax.div(j, 20) * 30 + 10 + lax.rem(j, 20))
        for j in (iota + 16 * u for u in range(5))
    )

    def macro(m, srcs):
        base = m * 80
        for u in range(5):
            v = plsc.load_gather(src_vmem, [srcs[u]])
            dst_vmem[pl.ds(base + 16 * u, 16)] = v
        return tuple(s + 120 for s in srcs)

    lax.fori_loop(0, WORDS_W // 80, macro, srcs)


def _body(pred_ref, tgt_ref, boxes_ref, obj_ref,
          p1, p2, tg, out1, out2, mbuf):
    wid = lax.axis_index("s") * NC + lax.axis_index("c")

    @pl.when(wid < NW)
    def _work():
        pltpu.sync_copy(pred_ref.at[pl.ds(PWORDS_W * wid, PWORDS_W)], p1)
        pltpu.sync_copy(
            pred_ref.at[pl.ds(CELLS // 2 * 30 + PWORDS_W * wid, PWORDS_W)], p2)
        pltpu.sync_copy(tgt_ref.at[pl.ds(TWORDS_W * wid, TWORDS_W)], tg)

        _copy_half(p1, out1)
        _copy_half(p2, out2)

        iota = lax.iota(jnp.int32, 16)
        fzero = jnp.zeros((16,), jnp.float32)
        step = jnp.full((16,), STEP, jnp.float32)
        expmask = _full(0x7FFFFFFF)
        inf_bits = _full(0x7F800000)
        one = _full(1)
        zero = _full(0)

        def cell_pipeline(t):
            q = lax.rem(CELLS_W * wid + t, 49)
            gi = lax.rem(q, 7).astype(jnp.float32)
            gj = lax.div(q, 7).astype(jnp.float32)
            tb = t * 30
            j0 = t * 10

            def gat_p(off):
                return plsc.load_gather(out1, [j0 + off])

            def gat_t(off):
                return plsc.load_gather(tg, [tb + off])

            def conv(x, y, w, h):
                cx = (x + gi) * step - w * 0.5
                cy = (y + gj) * step - h * 0.5
                return (jnp.maximum(cx, fzero), jnp.maximum(cy, fzero),
                        jnp.maximum(w, fzero), jnp.maximum(h, fzero))

            def iou(k):
                x1, y1, w1, h1 = conv(gat_p(5 * k), gat_p(5 * k + 1),
                                      gat_p(5 * k + 2), gat_p(5 * k + 3))
                x2, y2, w2, h2 = conv(gat_t(5 * k), gat_t(5 * k + 1),
                                      gat_t(5 * k + 2), gat_t(5 * k + 3))
                iw = w1 + w2 - (jnp.maximum(x1 + w1, x2 + w2)
                                - jnp.minimum(x1, x2))
                ih = h1 + h2 - (jnp.maximum(y1 + h1, y2 + h2)
                                - jnp.minimum(y1, y2))
                iw = jnp.maximum(iw, fzero)
                ih = jnp.maximum(ih, fzero)
                inter = iw * ih
                union = w1 * h1 + w2 * h2 - inter
                return inter / union

            iou0 = iou(0)
            iou1 = iou(1)
            nan0 = (plsc.bitcast(iou0, jnp.int32) & expmask) > inf_bits
            nan1 = (plsc.bitcast(iou1, jnp.int32) & expmask) > inf_bits
            maxi1 = (iou1 > iou0) | (nan1 & (~nan0))
            ioumax = jnp.where(maxi1, iou1, iou0)

            tc0 = gat_t(4)
            tc1 = gat_t(9)
            sig = tc1 > 4.0
            conf0 = jnp.where(sig, jnp.where(maxi1, fzero, ioumax), gat_p(4))
            conf1 = jnp.where(sig, jnp.where(maxi1, ioumax, fzero), gat_p(9))
            plsc.store_scatter(out1, [j0 + 4], conf0)
            plsc.store_scatter(out1, [j0 + 9], conf1)

            obj0 = jnp.where(tc0 > 4.0, one, zero)
            obj1 = jnp.where(sig, one, zero)
            objn0 = jnp.where(sig & maxi1, zero, obj0)
            objn1 = jnp.where(sig & (~maxi1), zero, obj1)
            plsc.store_scatter(mbuf, [2 * t], objn0)
            plsc.store_scatter(mbuf, [2 * t + 1], objn1)

        def group(g, carry):
            cell_pipeline(16 * g + iota)
            return carry

        lax.fori_loop(0, GROUPS, group, 0)

        pltpu.sync_copy(out1, boxes_ref.at[pl.ds(WORDS_W * wid, WORDS_W)])
        pltpu.sync_copy(out2,
                        boxes_ref.at[pl.ds(HALF + WORDS_W * wid, WORDS_W)])
        pltpu.sync_copy(mbuf,
                        obj_ref.at[pl.ds(2 * CELLS_W * wid, 2 * CELLS_W)])


_sc_call = functools.partial(
    pl.kernel,
    out_type=[
        jax.ShapeDtypeStruct((CELLS * 20,), jnp.float32),
        jax.ShapeDtypeStruct((2 * CELLS,), jnp.int32),
    ],
    mesh=_mesh,
    compiler_params=pltpu.CompilerParams(use_tc_tiling_on_sc=False,
                                         needs_layout_passes=False,
                                         skip_device_barrier=True),
    scratch_types=[
        pltpu.VMEM((PWORDS_W,), jnp.float32),
        pltpu.VMEM((PWORDS_W,), jnp.float32),
        pltpu.VMEM((TWORDS_W,), jnp.float32),
        pltpu.VMEM((WORDS_W,), jnp.float32),
        pltpu.VMEM((WORDS_W,), jnp.float32),
        pltpu.VMEM((2 * CELLS_W,), jnp.int32),
    ],
)(_body)


def kernel(pred, target):
    pf = pred.reshape(-1)
    tf = target.reshape(-1)
    boxes, objw = _sc_call(pf, tf)
    obj = objw.reshape(256, 7, 7, 2).astype(jnp.bool_)
    return (boxes.reshape(512, 7, 7, 2, 5), obj, ~obj)


# speedup vs baseline: 3.9874x; 3.9874x over previous
import functools

import jax
import jax.numpy as jnp
from jax import lax
from jax.experimental import pallas as pl
from jax.experimental.pallas import tpu as pltpu
from jax.experimental.pallas import tpu_sc as plsc

NC, NS = 2, 16
NW = 28
UNITS_W = 7
STEP = 1.0 / 7

_mesh = plsc.VectorSubcoreMesh(
    core_axis_name="c", subcore_axis_name="s", num_cores=NC, num_subcores=NS
)


def _full(v):
    return jnp.full((16,), v, jnp.int32)


def _body(pred_ref, tgt_ref, boxes_ref, obj_ref, noobj_ref,
          idxb, prows, tq, bx0, bx1, mbo, mbn, sem):
    wid = lax.axis_index("s") * NC + lax.axis_index("c")

    iota = lax.iota(jnp.int32, 16)
    fzero = jnp.zeros((16,), jnp.float32)
    step = jnp.full((16,), STEP, jnp.float32)
    expmask = _full(0x7FFFFFFF)
    inf_bits = _full(0x7F800000)
    one = _full(1)
    zero = _full(0)
    f0 = _full(0)

    @pl.when(wid < NW)
    def _work():
        def unit(u_, carry):
            u = UNITS_W * wid + u_
            f = lax.div(u, 4)
            q = lax.rem(u, 4)
            y = lax.div(f, 7)
            x = lax.rem(f, 7)
            qh = lax.div(q, 2)
            qo = lax.rem(q, 2) * 32

            for v in range(3):
                g = jnp.minimum(16 * v + iota, 39)
                h = lax.div(g, 20)
                r = lax.rem(g, 20)
                p = lax.div(r, 10)
                kc = lax.rem(r, 10)
                m = 490 * p + 10 * f + kc
                brow = lax.div(m, 20) * 30 + 10 + lax.rem(m, 20)
                idxb[pl.ds(16 * v, 16)] = brow * 4 + qh + 2 * h
            pltpu.async_copy(pred_ref.at[idxb], prows, sem).wait()
            pltpu.sync_copy(tgt_ref.at[pl.ds(30 * f, 10), pl.ds(64 * q, 64)],
                            tq)

            gi = x.astype(jnp.float32) + fzero
            gj = y.astype(jnp.float32) + fzero

            def conv(box):
                bx, by, bw, bh = box
                cx = (bx + gi) * step - bw * 0.5
                cy = (by + gj) * step - bh * 0.5
                return (jnp.maximum(cx, fzero), jnp.maximum(cy, fzero),
                        jnp.maximum(bw, fzero), jnp.maximum(bh, fzero))

            def iou(pb, tb):
                x1, y1, w1, h1 = conv(pb)
                x2, y2, w2, h2 = conv(tb)
                iw = w1 + w2 - (jnp.maximum(x1 + w1, x2 + w2)
                                - jnp.minimum(x1, x2))
                ih = h1 + h2 - (jnp.maximum(y1 + h1, y2 + h2)
                                - jnp.minimum(y1, y2))
                iw = jnp.maximum(iw, fzero)
                ih = jnp.maximum(ih, fzero)
                inter = iw * ih
                union = w1 * h1 + w2 * h2 - inter
                return inter / union

            for p in (0, 1):
                for s in (0, 1):
                    col = qo + 16 * s
                    bloc = 2 * (16 * s + iota) + p

                    def pld(h, k, c):
                        return prows[h * 20 + p * 10 + k * 5 + c,
                                     pl.ds(col, 16)]

                    def tld(off):
                        return plsc.load_gather(tq, [_full(off), bloc])

                    pb = {(k, c): pld(0, k, c)
                          for k in (0, 1) for c in range(5)}
                    iou0 = iou([pb[(0, c)] for c in range(4)],
                               [tld(c) for c in range(4)])
                    iou1 = iou([pb[(1, c)] for c in range(4)],
                               [tld(5 + c) for c in range(4)])
                    nan0 = (plsc.bitcast(iou0, jnp.int32) & expmask) > inf_bits
                    nan1 = (plsc.bitcast(iou1, jnp.int32) & expmask) > inf_bits
                    maxi1 = (iou1 > iou0) | (nan1 & (~nan0))
                    ioumax = jnp.where(maxi1, iou1, iou0)

                    tc0 = tld(4)
                    tc1 = tld(9)
                    sig = tc1 > 4.0
                    pb[(0, 4)] = jnp.where(
                        sig, jnp.where(maxi1, fzero, ioumax), pb[(0, 4)])
                    pb[(1, 4)] = jnp.where(
                        sig, jnp.where(maxi1, ioumax, fzero), pb[(1, 4)])

                    for k in (0, 1):
                        for c in range(5):
                            plsc.store_scatter(
                                bx0, [f0, f0, _full(c), f0, _full(k), bloc],
                                pb[(k, c)])
                            plsc.store_scatter(
                                bx1, [f0, f0, _full(c), f0, _full(k), bloc],
                                pld(1, k, c))

                    obj0 = jnp.where(tc0 > 4.0, one, zero)
                    obj1 = jnp.where(sig, one, zero)
                    objn0 = jnp.where(sig & maxi1, zero, obj0)
                    objn1 = jnp.where(sig & (~maxi1), zero, obj1)
                    plsc.store_scatter(mbo, [f0, f0, f0, bloc], objn0)
                    plsc.store_scatter(mbo, [f0, one, f0, bloc], objn1)
                    plsc.store_scatter(mbn, [f0, f0, f0, bloc], one - objn0)
                    plsc.store_scatter(mbn, [f0, one, f0, bloc], one - objn1)

            pltpu.sync_copy(
                bx0, boxes_ref.at[pl.ds(y, 1), pl.ds(x, 1), pl.ds(0, 5),
                                  pl.ds(qh, 1), pl.ds(0, 2),
                                  pl.ds(qo * 2, 64)])
            pltpu.sync_copy(
                bx1, boxes_ref.at[pl.ds(y, 1), pl.ds(x, 1), pl.ds(0, 5),
                                  pl.ds(2 + qh, 1), pl.ds(0, 2),
                                  pl.ds(qo * 2, 64)])
            pltpu.sync_copy(
                mbo, obj_ref.at[pl.ds(y, 1), pl.ds(0, 2), pl.ds(x, 1),
                                pl.ds(64 * q, 64)])
            pltpu.sync_copy(
                mbn, noobj_ref.at[pl.ds(y, 1), pl.ds(0, 2), pl.ds(x, 1),
                                  pl.ds(64 * q, 64)])
            return carry

        lax.fori_loop(0, UNITS_W, unit, 0)


_sc_call = functools.partial(
    pl.kernel,
    out_type=[
        jax.ShapeDtypeStruct((7, 7, 5, 4, 2, 128), jnp.float32),
        jax.ShapeDtypeStruct((7, 2, 7, 256), jnp.int32),
        jax.ShapeDtypeStruct((7, 2, 7, 256), jnp.int32),
    ],
    mesh=_mesh,
    compiler_params=pltpu.CompilerParams(use_tc_tiling_on_sc=False,
                                         needs_layout_passes=False),
    scratch_types=[
        pltpu.VMEM((48,), jnp.int32),
        pltpu.VMEM((48, 64), jnp.float32),
        pltpu.VMEM((10, 64), jnp.float32),
        pltpu.VMEM((1, 1, 5, 1, 2, 64), jnp.float32),
        pltpu.VMEM((1, 1, 5, 1, 2, 64), jnp.float32),
        pltpu.VMEM((1, 2, 1, 64), jnp.int32),
        pltpu.VMEM((1, 2, 1, 64), jnp.int32),
        pltpu.SemaphoreType.DMA,
    ],
)(_body)


def kernel(pred, target):
    pt = jnp.transpose(pred, (1, 2, 3, 0)).reshape(5880, 64)
    tt = jnp.transpose(target, (1, 2, 3, 0)).reshape(1470, 256)
    boxes6, mo, mn = _sc_call(pt, tt)
    boxes = boxes6.transpose((3, 5, 0, 1, 4, 2)).reshape(512, 7, 7, 2, 5)
    obj = mo.transpose((3, 0, 2, 1)).astype(jnp.bool_)
    noobj = mn.transpose((3, 0, 2, 1)).astype(jnp.bool_)
    return (boxes, obj, noobj)


# speedup vs baseline: 4.3161x; 1.0824x over previous
import functools

import jax
import jax.numpy as jnp
from jax import lax
from jax.experimental import pallas as pl
from jax.experimental.pallas import tpu as pltpu
from jax.experimental.pallas import tpu_sc as plsc

NC, NS = 2, 16
NW = 28
UNITS_W = 7
STEP = 1.0 / 7

_mesh = plsc.VectorSubcoreMesh(
    core_axis_name="c", subcore_axis_name="s", num_cores=NC, num_subcores=NS
)


def _full(v):
    return jnp.full((16,), v, jnp.int32)


def _body(pred_ref, tgt_ref, boxes_ref, obj_ref, noobj_ref,
          idxb, prows, tq, bx0, bx1, mbo, mbn, sem):
    wid = lax.axis_index("s") * NC + lax.axis_index("c")

    iota = lax.iota(jnp.int32, 16)
    fzero = jnp.zeros((16,), jnp.float32)
    step = jnp.full((16,), STEP, jnp.float32)
    expmask = _full(0x7FFFFFFF)
    inf_bits = _full(0x7F800000)
    one = _full(1)
    zero = _full(0)
    f0 = _full(0)

    @pl.when(wid < NW)
    def _work():
        def unit(u_, carry):
            u = UNITS_W * wid + u_
            f = lax.div(u, 4)
            q = lax.rem(u, 4)
            y = lax.div(f, 7)
            x = lax.rem(f, 7)
            qh = lax.div(q, 2)
            qo = lax.rem(q, 2) * 32

            for v in range(3):
                g = jnp.minimum(16 * v + iota, 39)
                h = lax.div(g, 20)
                r = lax.rem(g, 20)
                p = lax.div(r, 10)
                kc = lax.rem(r, 10)
                m = 490 * p + 10 * f + kc
                brow = lax.div(m, 20) * 30 + 10 + lax.rem(m, 20)
                idxb[pl.ds(16 * v, 16)] = brow * 4 + qh + 2 * h
            tidx = (30 * f + jnp.minimum(iota, 9)) * 4 + q
            idxb[pl.ds(48, 16)] = tidx
            pred_cp = pltpu.async_copy(pred_ref.at[idxb.at[pl.ds(0, 48)]],
                                       prows, sem)
            tgt_cp = pltpu.async_copy(tgt_ref.at[idxb.at[pl.ds(48, 16)]],
                                      tq, sem)
            pred_cp.wait()
            tgt_cp.wait()

            gi = x.astype(jnp.float32) + fzero
            gj = y.astype(jnp.float32) + fzero

            def conv(box):
                bx, by, bw, bh = box
                cx = (bx + gi) * step - bw * 0.5
                cy = (by + gj) * step - bh * 0.5
                return (jnp.maximum(cx, fzero), jnp.maximum(cy, fzero),
                        jnp.maximum(bw, fzero), jnp.maximum(bh, fzero))

            def iou(pb, tb):
                x1, y1, w1, h1 = conv(pb)
                x2, y2, w2, h2 = conv(tb)
                iw = w1 + w2 - (jnp.maximum(x1 + w1, x2 + w2)
                                - jnp.minimum(x1, x2))
                ih = h1 + h2 - (jnp.maximum(y1 + h1, y2 + h2)
                                - jnp.minimum(y1, y2))
                iw = jnp.maximum(iw, fzero)
                ih = jnp.maximum(ih, fzero)
                inter = iw * ih
                union = w1 * h1 + w2 * h2 - inter
                return inter / union

            for p in (0, 1):
                for s in (0, 1):
                    col = qo + 16 * s
                    bloc = 2 * (16 * s + iota) + p

                    def pld(h, k, c):
                        return prows[h * 20 + p * 10 + k * 5 + c,
                                     pl.ds(col, 16)]

                    def tld(off):
                        return plsc.load_gather(tq, [_full(off), bloc])

                    pb = {(k, c): pld(0, k, c)
                          for k in (0, 1) for c in range(5)}
                    iou0 = iou([pb[(0, c)] for c in range(4)],
                               [tld(c) for c in range(4)])
                    iou1 = iou([pb[(1, c)] for c in range(4)],
                               [tld(5 + c) for c in range(4)])
                    nan0 = (plsc.bitcast(iou0, jnp.int32) & expmask) > inf_bits
                    nan1 = (plsc.bitcast(iou1, jnp.int32) & expmask) > inf_bits
                    maxi1 = (iou1 > iou0) | (nan1 & (~nan0))
                    ioumax = jnp.where(maxi1, iou1, iou0)

                    tc0 = tld(4)
                    tc1 = tld(9)
                    sig = tc1 > 4.0
                    pb[(0, 4)] = jnp.where(
                        sig, jnp.where(maxi1, fzero, ioumax), pb[(0, 4)])
                    pb[(1, 4)] = jnp.where(
                        sig, jnp.where(maxi1, ioumax, fzero), pb[(1, 4)])

                    for k in (0, 1):
                        for c in range(5):
                            plsc.store_scatter(
                                bx0, [f0, f0, _full(c), f0, _full(k), bloc],
                                pb[(k, c)])
                            plsc.store_scatter(
                                bx1, [f0, f0, _full(c), f0, _full(k), bloc],
                                pld(1, k, c))

                    obj0 = jnp.where(tc0 > 4.0, one, zero)
                    obj1 = jnp.where(sig, one, zero)
                    objn0 = jnp.where(sig & maxi1, zero, obj0)
                    objn1 = jnp.where(sig & (~maxi1), zero, obj1)
                    plsc.store_scatter(mbo, [f0, f0, f0, bloc], objn0)
                    plsc.store_scatter(mbo, [f0, one, f0, bloc], objn1)
                    plsc.store_scatter(mbn, [f0, f0, f0, bloc], one - objn0)
                    plsc.store_scatter(mbn, [f0, one, f0, bloc], one - objn1)

            pltpu.sync_copy(
                bx0, boxes_ref.at[pl.ds(y, 1), pl.ds(x, 1), pl.ds(0, 5),
                                  pl.ds(qh, 1), pl.ds(0, 2),
                                  pl.ds(qo * 2, 64)])
            pltpu.sync_copy(
                bx1, boxes_ref.at[pl.ds(y, 1), pl.ds(x, 1), pl.ds(0, 5),
                                  pl.ds(2 + qh, 1), pl.ds(0, 2),
                                  pl.ds(qo * 2, 64)])
            pltpu.sync_copy(
                mbo, obj_ref.at[pl.ds(y, 1), pl.ds(0, 2), pl.ds(x, 1),
                                pl.ds(64 * q, 64)])
            pltpu.sync_copy(
                mbn, noobj_ref.at[pl.ds(y, 1), pl.ds(0, 2), pl.ds(x, 1),
                                  pl.ds(64 * q, 64)])
            return carry

        lax.fori_loop(0, UNITS_W, unit, 0)


_sc_call = functools.partial(
    pl.kernel,
    out_type=[
        jax.ShapeDtypeStruct((7, 7, 5, 4, 2, 128), jnp.float32),
        jax.ShapeDtypeStruct((7, 2, 7, 256), jnp.int32),
        jax.ShapeDtypeStruct((7, 2, 7, 256), jnp.int32),
    ],
    mesh=_mesh,
    compiler_params=pltpu.CompilerParams(use_tc_tiling_on_sc=False,
                                         needs_layout_passes=False),
    scratch_types=[
        pltpu.VMEM((64,), jnp.int32),
        pltpu.VMEM((48, 64), jnp.float32),
        pltpu.VMEM((16, 64), jnp.float32),
        pltpu.VMEM((1, 1, 5, 1, 2, 64), jnp.float32),
        pltpu.VMEM((1, 1, 5, 1, 2, 64), jnp.float32),
        pltpu.VMEM((1, 2, 1, 64), jnp.int32),
        pltpu.VMEM((1, 2, 1, 64), jnp.int32),
        pltpu.SemaphoreType.DMA,
    ],
)(_body)


def kernel(pred, target):
    pt = jnp.transpose(pred, (1, 2, 3, 0)).reshape(5880, 64)
    tt = jnp.transpose(target, (1, 2, 3, 0)).reshape(5880, 64)
    boxes6, mo, mn = _sc_call(pt, tt)
    boxes = boxes6.transpose((3, 5, 0, 1, 4, 2)).reshape(512, 7, 7, 2, 5)
    obj = mo.transpose((3, 0, 2, 1)).astype(jnp.bool_)
    noobj = mn.transpose((3, 0, 2, 1)).astype(jnp.bool_)
    return (boxes, obj, noobj)


# speedup vs baseline: 4.4741x; 1.0366x over previous
import functools

import jax
import jax.numpy as jnp
from jax import lax
from jax.experimental import pallas as pl
from jax.experimental.pallas import tpu as pltpu
from jax.experimental.pallas import tpu_sc as plsc

NC, NS = 2, 16
NW = 28
UNITS_W = 7
STEP = 1.0 / 7

_mesh = plsc.VectorSubcoreMesh(
    core_axis_name="c", subcore_axis_name="s", num_cores=NC, num_subcores=NS
)


def _full(v):
    return jnp.full((16,), v, jnp.int32)


def _body(pred_ref, tgt_ref, boxes_ref, masks_ref,
          idxb, prows, tq, bx0, bx1, mb, sem):
    wid = lax.axis_index("s") * NC + lax.axis_index("c")

    iota = lax.iota(jnp.int32, 16)
    fzero = jnp.zeros((16,), jnp.float32)
    step = jnp.full((16,), STEP, jnp.float32)
    expmask = _full(0x7FFFFFFF)
    inf_bits = _full(0x7F800000)
    one = _full(1)
    zero = _full(0)
    f0 = _full(0)

    @pl.when(wid < NW)
    def _work():
        def unit(u_, carry):
            u = UNITS_W * wid + u_
            f = lax.div(u, 4)
            q = lax.rem(u, 4)
            y = lax.div(f, 7)
            x = lax.rem(f, 7)
            qh = lax.div(q, 2)
            qo = lax.rem(q, 2) * 32

            for v in range(3):
                g = jnp.minimum(16 * v + iota, 39)
                h = lax.div(g, 20)
                r = lax.rem(g, 20)
                p = lax.div(r, 10)
                kc = lax.rem(r, 10)
                m = 490 * p + 10 * f + kc
                brow = lax.div(m, 20) * 30 + 10 + lax.rem(m, 20)
                idxb[pl.ds(16 * v, 16)] = brow * 4 + qh + 2 * h
            tidx = (30 * f + jnp.minimum(iota, 9)) * 4 + q
            idxb[pl.ds(48, 16)] = tidx
            pred_cp = pltpu.async_copy(pred_ref.at[idxb.at[pl.ds(0, 48)]],
                                       prows, sem)
            tgt_cp = pltpu.async_copy(tgt_ref.at[idxb.at[pl.ds(48, 16)]],
                                      tq, sem)
            pred_cp.wait()
            tgt_cp.wait()

            gi = x.astype(jnp.float32) + fzero
            gj = y.astype(jnp.float32) + fzero

            def conv(box):
                bx, by, bw, bh = box
                cx = (bx + gi) * step - bw * 0.5
                cy = (by + gj) * step - bh * 0.5
                return (jnp.maximum(cx, fzero), jnp.maximum(cy, fzero),
                        jnp.maximum(bw, fzero), jnp.maximum(bh, fzero))

            def iou(pb, tb):
                x1, y1, w1, h1 = conv(pb)
                x2, y2, w2, h2 = conv(tb)
                iw = w1 + w2 - (jnp.maximum(x1 + w1, x2 + w2)
                                - jnp.minimum(x1, x2))
                ih = h1 + h2 - (jnp.maximum(y1 + h1, y2 + h2)
                                - jnp.minimum(y1, y2))
                iw = jnp.maximum(iw, fzero)
                ih = jnp.maximum(ih, fzero)
                inter = iw * ih
                union = w1 * h1 + w2 * h2 - inter
                return inter / union

            for p in (0, 1):
                for s in (0, 1):
                    col = qo + 16 * s
                    bloc = 2 * (16 * s + iota) + p

                    def pld(h, k, c):
                        return prows[h * 20 + p * 10 + k * 5 + c,
                                     pl.ds(col, 16)]

                    def tld(off):
                        return plsc.load_gather(tq, [_full(off), bloc])

                    pb = {(k, c): pld(0, k, c)
                          for k in (0, 1) for c in range(5)}
                    iou0 = iou([pb[(0, c)] for c in range(4)],
                               [tld(c) for c in range(4)])
                    iou1 = iou([pb[(1, c)] for c in range(4)],
                               [tld(5 + c) for c in range(4)])
                    nan0 = (plsc.bitcast(iou0, jnp.int32) & expmask) > inf_bits
                    nan1 = (plsc.bitcast(iou1, jnp.int32) & expmask) > inf_bits
                    maxi1 = (iou1 > iou0) | (nan1 & (~nan0))
                    ioumax = jnp.where(maxi1, iou1, iou0)

                    tc0 = tld(4)
                    tc1 = tld(9)
                    sig = tc1 > 4.0
                    pb[(0, 4)] = jnp.where(
                        sig, jnp.where(maxi1, fzero, ioumax), pb[(0, 4)])
                    pb[(1, 4)] = jnp.where(
                        sig, jnp.where(maxi1, ioumax, fzero), pb[(1, 4)])

                    for k in (0, 1):
                        for c in range(5):
                            plsc.store_scatter(
                                bx0, [f0, f0, _full(c), f0, _full(k), bloc],
                                pb[(k, c)])
                            plsc.store_scatter(
                                bx1, [f0, f0, _full(c), f0, _full(k), bloc],
                                pld(1, k, c))

                    obj0 = jnp.where(tc0 > 4.0, one, zero)
                    obj1 = jnp.where(sig, one, zero)
                    objn0 = jnp.where(sig & maxi1, zero, obj0)
                    objn1 = jnp.where(sig & (~maxi1), zero, obj1)
                    plsc.store_scatter(mb, [f0, f0, f0, f0, bloc], objn0)
                    plsc.store_scatter(mb, [f0, f0, one, f0, bloc], objn1)
                    plsc.store_scatter(mb, [one, f0, f0, f0, bloc],
                                       one - objn0)
                    plsc.store_scatter(mb, [one, f0, one, f0, bloc],
                                       one - objn1)

            pltpu.sync_copy(
                bx0, boxes_ref.at[pl.ds(y, 1), pl.ds(x, 1), pl.ds(0, 5),
                                  pl.ds(qh, 1), pl.ds(0, 2),
                                  pl.ds(qo * 2, 64)])
            pltpu.sync_copy(
                bx1, boxes_ref.at[pl.ds(y, 1), pl.ds(x, 1), pl.ds(0, 5),
                                  pl.ds(2 + qh, 1), pl.ds(0, 2),
                                  pl.ds(qo * 2, 64)])
            pltpu.sync_copy(
                mb, masks_ref.at[pl.ds(0, 2), pl.ds(y, 1), pl.ds(0, 2),
                                 pl.ds(x, 1), pl.ds(64 * q, 64)])
            return carry

        lax.fori_loop(0, UNITS_W, unit, 0)


_sc_call = functools.partial(
    pl.kernel,
    out_type=[
        jax.ShapeDtypeStruct((7, 7, 5, 4, 2, 128), jnp.float32),
        jax.ShapeDtypeStruct((2, 7, 2, 7, 256), jnp.int32),
    ],
    mesh=_mesh,
    compiler_params=pltpu.CompilerParams(use_tc_tiling_on_sc=False,
                                         needs_layout_passes=False),
    scratch_types=[
        pltpu.VMEM((64,), jnp.int32),
        pltpu.VMEM((48, 64), jnp.float32),
        pltpu.VMEM((16, 64), jnp.float32),
        pltpu.VMEM((1, 1, 5, 1, 2, 64), jnp.float32),
        pltpu.VMEM((1, 1, 5, 1, 2, 64), jnp.float32),
        pltpu.VMEM((2, 1, 2, 1, 64), jnp.int32),
        pltpu.SemaphoreType.DMA,
    ],
)(_body)


def kernel(pred, target):
    pt = jnp.transpose(pred, (1, 2, 3, 0)).reshape(5880, 64)
    tt = jnp.transpose(target, (1, 2, 3, 0)).reshape(5880, 64)
    boxes6, m2 = _sc_call(pt, tt)
    boxes = boxes6.transpose((3, 5, 0, 1, 4, 2)).reshape(512, 7, 7, 2, 5)
    mboth = m2.transpose((0, 4, 1, 3, 2)).astype(jnp.bool_)
    return (boxes, mboth[0], mboth[1])


# speedup vs baseline: 5.2583x; 1.1753x over previous
import functools

import jax
import jax.numpy as jnp
from jax import lax
from jax.experimental import pallas as pl
from jax.experimental.pallas import tpu as pltpu
from jax.experimental.pallas import tpu_sc as plsc

NC, NS = 2, 16
NW = 28
UNITS_W = 7
STEP = 1.0 / 7

_mesh = plsc.VectorSubcoreMesh(
    core_axis_name="c", subcore_axis_name="s", num_cores=NC, num_subcores=NS
)


def _full(v):
    return jnp.full((16,), v, jnp.int32)


def _body(pred_ref, tgt_ref, boxes_ref, masks_ref,
          idxb, prows, tq, bx0, bx1, mb, sem):
    wid = lax.axis_index("s") * NC + lax.axis_index("c")

    iota = lax.iota(jnp.int32, 16)
    fzero = jnp.zeros((16,), jnp.float32)
    step = jnp.full((16,), STEP, jnp.float32)
    expmask = _full(0x7FFFFFFF)
    inf_bits = _full(0x7F800000)
    one = _full(1)
    zero = _full(0)
    f0 = _full(0)

    @pl.when(wid < NW)
    def _work():
        def prefetch(u_, carry):
            u = UNITS_W * wid + u_
            f = lax.div(u, 4)
            q = lax.rem(u, 4)
            qh = lax.div(q, 2)
            for v in range(3):
                g = jnp.minimum(16 * v + iota, 39)
                h = lax.div(g, 20)
                r = lax.rem(g, 20)
                p = lax.div(r, 10)
                kc = lax.rem(r, 10)
                m = 490 * p + 10 * f + kc
                brow = lax.div(m, 20) * 30 + 10 + lax.rem(m, 20)
                idxb[u_, pl.ds(16 * v, 16)] = brow * 4 + qh + 2 * h
            idxb[u_, pl.ds(48, 16)] = (30 * f + jnp.minimum(iota, 9)) * 4 + q
            pltpu.async_copy(pred_ref.at[idxb.at[u_, pl.ds(0, 48)]],
                             prows.at[u_], sem.at[u_, 0])
            pltpu.async_copy(tgt_ref.at[idxb.at[u_, pl.ds(48, 16)]],
                             tq.at[u_], sem.at[u_, 1])
            return carry

        lax.fori_loop(0, UNITS_W, prefetch, 0)

        def unit(u_, carry):
            u = UNITS_W * wid + u_
            f = lax.div(u, 4)
            q = lax.rem(u, 4)
            y = lax.div(f, 7)
            x = lax.rem(f, 7)
            qh = lax.div(q, 2)
            qo = lax.rem(q, 2) * 32

            pltpu.make_async_copy(pred_ref.at[idxb.at[u_, pl.ds(0, 48)]],
                                  prows.at[u_], sem.at[u_, 0]).wait()
            pltpu.make_async_copy(tgt_ref.at[idxb.at[u_, pl.ds(48, 16)]],
                                  tq.at[u_], sem.at[u_, 1]).wait()

            gi = x.astype(jnp.float32) + fzero
            gj = y.astype(jnp.float32) + fzero

            def conv(box):
                bx, by, bw, bh = box
                cx = (bx + gi) * step - bw * 0.5
                cy = (by + gj) * step - bh * 0.5
                return (jnp.maximum(cx, fzero), jnp.maximum(cy, fzero),
                        jnp.maximum(bw, fzero), jnp.maximum(bh, fzero))

            def iou(pb, tb):
                x1, y1, w1, h1 = conv(pb)
                x2, y2, w2, h2 = conv(tb)
                iw = w1 + w2 - (jnp.maximum(x1 + w1, x2 + w2)
                                - jnp.minimum(x1, x2))
                ih = h1 + h2 - (jnp.maximum(y1 + h1, y2 + h2)
                                - jnp.minimum(y1, y2))
                iw = jnp.maximum(iw, fzero)
                ih = jnp.maximum(ih, fzero)
                inter = iw * ih
                union = w1 * h1 + w2 * h2 - inter
                return inter / union

            for p in (0, 1):
                for s in (0, 1):
                    col = qo + 16 * s
                    bloc = 2 * (16 * s + iota) + p

                    def pld(h, k, c):
                        return prows[u_, h * 20 + p * 10 + k * 5 + c,
                                     pl.ds(col, 16)]

                    def tld(off):
                        return plsc.load_gather(tq.at[u_],
                                                [_full(off), bloc])

                    pb = {(k, c): pld(0, k, c)
                          for k in (0, 1) for c in range(5)}
                    iou0 = iou([pb[(0, c)] for c in range(4)],
                               [tld(c) for c in range(4)])
                    iou1 = iou([pb[(1, c)] for c in range(4)],
                               [tld(5 + c) for c in range(4)])
                    nan0 = (plsc.bitcast(iou0, jnp.int32) & expmask) > inf_bits
                    nan1 = (plsc.bitcast(iou1, jnp.int32) & expmask) > inf_bits
                    maxi1 = (iou1 > iou0) | (nan1 & (~nan0))
                    ioumax = jnp.where(maxi1, iou1, iou0)

                    tc0 = tld(4)
                    tc1 = tld(9)
                    sig = tc1 > 4.0
                    pb[(0, 4)] = jnp.where(
                        sig, jnp.where(maxi1, fzero, ioumax), pb[(0, 4)])
                    pb[(1, 4)] = jnp.where(
                        sig, jnp.where(maxi1, ioumax, fzero), pb[(1, 4)])

                    for k in (0, 1):
                        for c in range(5):
                            plsc.store_scatter(
                                bx0, [f0, f0, _full(c), f0, _full(k), bloc],
                                pb[(k, c)])
                            plsc.store_scatter(
                                bx1, [f0, f0, _full(c), f0, _full(k), bloc],
                                pld(1, k, c))

                    obj0 = jnp.where(tc0 > 4.0, one, zero)
                    obj1 = jnp.where(sig, one, zero)
                    objn0 = jnp.where(sig & maxi1, zero, obj0)
                    objn1 = jnp.where(sig & (~maxi1), zero, obj1)
                    plsc.store_scatter(mb, [f0, f0, f0, f0, bloc], objn0)
                    plsc.store_scatter(mb, [f0, f0, one, f0, bloc], objn1)
                    plsc.store_scatter(mb, [one, f0, f0, f0, bloc],
                                       one - objn0)
                    plsc.store_scatter(mb, [one, f0, one, f0, bloc],
                                       one - objn1)

            pltpu.sync_copy(
                bx0, boxes_ref.at[pl.ds(y, 1), pl.ds(x, 1), pl.ds(0, 5),
                                  pl.ds(qh, 1), pl.ds(0, 2),
                                  pl.ds(qo * 2, 64)])
            pltpu.sync_copy(
                bx1, boxes_ref.at[pl.ds(y, 1), pl.ds(x, 1), pl.ds(0, 5),
                                  pl.ds(2 + qh, 1), pl.ds(0, 2),
                                  pl.ds(qo * 2, 64)])
            pltpu.sync_copy(
                mb, masks_ref.at[pl.ds(0, 2), pl.ds(y, 1), pl.ds(0, 2),
                                 pl.ds(x, 1), pl.ds(64 * q, 64)])
            return carry

        lax.fori_loop(0, UNITS_W, unit, 0)


_sc_call = functools.partial(
    pl.kernel,
    out_type=[
        jax.ShapeDtypeStruct((7, 7, 5, 4, 2, 128), jnp.float32),
        jax.ShapeDtypeStruct((2, 7, 2, 7, 256), jnp.int32),
    ],
    mesh=_mesh,
    compiler_params=pltpu.CompilerParams(use_tc_tiling_on_sc=False,
                                         needs_layout_passes=False),
    scratch_types=[
        pltpu.VMEM((7, 64), jnp.int32),
        pltpu.VMEM((7, 48, 64), jnp.float32),
        pltpu.VMEM((7, 16, 64), jnp.float32),
        pltpu.VMEM((1, 1, 5, 1, 2, 64), jnp.float32),
        pltpu.VMEM((1, 1, 5, 1, 2, 64), jnp.float32),
        pltpu.VMEM((2, 1, 2, 1, 64), jnp.int32),
        pltpu.SemaphoreType.DMA((7, 2)),
    ],
)(_body)


def kernel(pred, target):
    pt = jnp.transpose(pred, (1, 2, 3, 0)).reshape(5880, 64)
    tt = jnp.transpose(target, (1, 2, 3, 0)).reshape(5880, 64)
    boxes6, m2 = _sc_call(pt, tt)
    boxes = boxes6.transpose((3, 5, 0, 1, 4, 2)).reshape(512, 7, 7, 2, 5)
    mboth = m2.transpose((0, 4, 1, 3, 2)).astype(jnp.bool_)
    return (boxes, mboth[0], mboth[1])
